# baseline (speedup 1.0000x reference)
import functools

import jax
import jax.numpy as jnp
from jax import lax
from jax.experimental import pallas as pl
from jax.experimental.pallas import tpu as pltpu
from jax.experimental.pallas import tpu_sc as plsc

N = 100000
E = 3200000
D = 16
NC = 2
NS = 16
L = 16
NW = NC * NS
NPAD = 102400
SEG = NPAD // NS
CH = 128
NCHUNK = E // CH
FULL = NCHUNK // NW
TAIL = NCHUNK - FULL * NW
G = 11
NG = FULL // G
NB = 4
GM = 5
NGM = (FULL - 1) // GM
RB = 1024
RB2 = 1000

_MESH = dict(core_axis_name="c", subcore_axis_name="s")


def _wid():
    return lax.axis_index("s") * NC + lax.axis_index("c")


@functools.partial(
    pl.kernel,
    out_type=jax.ShapeDtypeStruct((NC, NPAD), jnp.float32),
    mesh=plsc.VectorSubcoreMesh(**_MESH),
    compiler_params=pltpu.CompilerParams(use_tc_tiling_on_sc=False),
    scratch_types=(
        [pltpu.VMEM_SHARED((NPAD,), jnp.float32),
         pltpu.VMEM((SEG,), jnp.float32),
         pltpu.VMEM((CH,), jnp.float32)]
        + [pltpu.VMEM((G, CH), jnp.int32) for _ in range(NB)]
        + [pltpu.SemaphoreType.DMA for _ in range(2 * NB)]
    ),
)
def _hist_kernel(row2d_hbm, hist_out, hist_sm, zbuf, ones, *rest):
    ri = rest[0:NB]
    si = rest[NB:2 * NB]
    sd = rest[2 * NB:3 * NB]
    cid = lax.axis_index("c")
    sid = lax.axis_index("s")
    wid = _wid()
    tb = wid * FULL

    zv = jnp.zeros((L,), jnp.float32)

    def zfill(i, _):
        zbuf[pl.ds(i * L, L)] = zv
        return 0

    lax.fori_loop(0, SEG // L, zfill, 0)

    ov = jnp.ones((L,), jnp.float32)

    def ofill(i, _):
        ones[pl.ds(i * L, L)] = ov
        return 0

    lax.fori_loop(0, CH // L, ofill, 0)

    pltpu.sync_copy(zbuf, hist_sm.at[pl.ds(sid * SEG, SEG)])
    plsc.subcore_barrier()

    def stage(g, s):
        pltpu.async_copy(row2d_hbm.at[pl.ds(tb + g * G, G), :], ri[s], si[s])

    def wait_stage(g, s):
        pltpu.make_async_copy(
            row2d_hbm.at[pl.ds(tb + g * G, G), :], ri[s], si[s]).wait()

    def fire(g, s):
        for b in range(G):
            pltpu.async_copy(ones, hist_sm.at[ri[s].at[b]], sd[s], add=True)

    def drain(s):
        for b in range(G):
            pltpu.make_async_copy(ones, hist_sm.at[ri[s].at[b]], sd[s]).wait()

    def turn(g, s, do_drain=True, stage_ahead=True):
        if do_drain:
            drain((s + 2) % NB)
        if stage_ahead:
            stage(g + 2, (s + 2) % NB)
        wait_stage(g, s)
        fire(g, s)

    stage(0, 0)
    stage(1, 1)
    turn(0, 0, do_drain=False)
    turn(1, 1, do_drain=False)
    turn(2, 2)
    turn(3, 3)

    def quad(q, _):
        g = q * NB
        for j in range(NB):
            turn(g + j, j)
        return 0

    lax.fori_loop(1, (NG - 3) // NB, quad, 0)
    turn(NG - 3, (NG - 3) % NB)
    turn(NG - 2, (NG - 2) % NB, stage_ahead=False)
    turn(NG - 1, (NG - 1) % NB, stage_ahead=False)
    drain((NG - 2) % NB)
    drain((NG - 1) % NB)

    @pl.when(wid < TAIL)
    def _tail():
        c0 = NW * FULL + wid
        pltpu.sync_copy(row2d_hbm.at[pl.ds(c0, 1), :], ri[0].at[pl.ds(0, 1), :])
        pltpu.sync_copy(ones, hist_sm.at[ri[0].at[0]], add=True)

    plsc.subcore_barrier()
    pltpu.sync_copy(hist_sm.at[pl.ds(sid * SEG, SEG)],
                    hist_out.at[cid, pl.ds(sid * SEG, SEG)])


def _rsqrt(x):
    i = lax.bitcast_convert_type(x, jnp.int32)
    i = 0x5F3759DF - lax.shift_right_logical(i, 1)
    y = lax.bitcast_convert_type(i, jnp.float32)
    for _ in range(3):
        y = y * (1.5 - 0.5 * x * y * y)
    return y


RPT = NPAD // NW
LASTV = N - (NW - 1) * RPT


@functools.partial(
    pl.kernel,
    out_type=(jax.ShapeDtypeStruct((NPAD,), jnp.float32),
              jax.ShapeDtypeStruct((N, D), jnp.float32)),
    mesh=plsc.VectorSubcoreMesh(**_MESH),
    compiler_params=pltpu.CompilerParams(use_tc_tiling_on_sc=False),
    scratch_types=[
        pltpu.VMEM((RPT,), jnp.float32),
        pltpu.VMEM((RPT,), jnp.float32),
        pltpu.VMEM((RPT,), jnp.float32),
        pltpu.VMEM((RPT, D), jnp.float32),
    ],
)
def _scale_sc(hist_hbm, emb_hbm, dis_out, embs_out, h0, h1, disv, embv):
    wid = _wid()
    base = wid * RPT
    pltpu.sync_copy(hist_hbm.at[0, pl.ds(base, RPT)], h0)
    pltpu.sync_copy(hist_hbm.at[1, pl.ds(base, RPT)], h1)

    def dbody(i, _):
        deg = h0[pl.ds(i * L, L)] + h1[pl.ds(i * L, L)] + 1.0
        disv[pl.ds(i * L, L)] = _rsqrt(deg)
        return 0

    lax.fori_loop(0, RPT // L, dbody, 0)
    pltpu.sync_copy(disv, dis_out.at[pl.ds(base, RPT)])

    def scale_rows(nrows):
        pltpu.sync_copy(emb_hbm.at[pl.ds(base, nrows)],
                        embv.at[pl.ds(0, nrows)])

        def rbody(i, _):
            dv = disv[pl.ds(i * L, L)]
            for j in range(L):
                r = i * L + j
                dspl = jnp.broadcast_to(dv[j], (L,))
                embv[r, :] = embv[r, :] * dspl
            return 0

        lax.fori_loop(0, nrows // L, rbody, 0)
        pltpu.sync_copy(embv.at[pl.ds(0, nrows)],
                        embs_out.at[pl.ds(base, nrows)])

    @pl.when(wid < NW - 1)
    def _full():
        scale_rows(RPT)

    @pl.when(wid == NW - 1)
    def _last():
        scale_rows(LASTV)


@functools.partial(
    pl.kernel,
    out_type=jax.ShapeDtypeStruct((NC, NPAD, D), jnp.float32),
    mesh=plsc.VectorSubcoreMesh(**_MESH),
    compiler_params=pltpu.CompilerParams(use_tc_tiling_on_sc=False),
    scratch_types=(
        [pltpu.VMEM_SHARED((NPAD, D), jnp.float32)]
        + [pltpu.VMEM((GM, CH), jnp.int32) for _ in range(8)]
        + [pltpu.VMEM((GM, CH, D), jnp.float32) for _ in range(2)]
        + [pltpu.SemaphoreType.DMA for _ in range(8)]
    ),
)
def _msg_kernel(row2d_hbm, col2d_hbm, embs_hbm, m_out, m_sm, *rest):
    ri = rest[0:4]
    ci = rest[4:8]
    rw = rest[8:10]
    si = rest[10:14]
    sc = rest[14:16]
    sd = rest[16:18]
    cid = lax.axis_index("c")
    sid = lax.axis_index("s")
    wid = _wid()
    tb = wid * FULL

    @pl.when(cid == 0)
    def _init0():
        nv = N // NS
        pltpu.sync_copy(embs_hbm.at[pl.ds(sid * nv, nv)],
                        m_sm.at[pl.ds(sid * nv, nv)])

    @pl.when(cid == 1)
    def _init1():
        zv = jnp.zeros((L,), jnp.float32)
        for b in range(GM):
            def zfill(r, _, _b=b):
                rw[0][_b, r, :] = zv
                return 0
            lax.fori_loop(0, CH, zfill, 0)
        def zcopy(j, _):
            pltpu.async_copy(rw[0].at[0],
                             m_sm.at[pl.ds(sid * SEG + j * CH, CH)], sd[0])
            return 0

        lax.fori_loop(0, SEG // CH, zcopy, 0)

        def zwait(j, _):
            pltpu.make_async_copy(
                rw[0].at[0], m_sm.at[pl.ds(sid * SEG + j * CH, CH)],
                sd[0]).wait()
            return 0

        lax.fori_loop(0, SEG // CH, zwait, 0)

    plsc.subcore_barrier()

    def stage(g, s4):
        c0 = tb + g * GM
        pltpu.async_copy(row2d_hbm.at[pl.ds(c0, GM), :], ri[s4], si[s4])
        pltpu.async_copy(col2d_hbm.at[pl.ds(c0, GM), :], ci[s4], si[s4])

    def wait_stage(g, s4):
        c0 = tb + g * GM
        pltpu.make_async_copy(row2d_hbm.at[pl.ds(c0, GM), :], ri[s4],
                              si[s4]).wait()
        pltpu.make_async_copy(col2d_hbm.at[pl.ds(c0, GM), :], ci[s4],
                              si[s4]).wait()

    def fire(s4, s2):
        for b in range(GM):
            pltpu.async_copy(embs_hbm.at[ri[s4].at[b]], rw[s2].at[b], sc[s2])
        for b in range(GM):
            pltpu.make_async_copy(embs_hbm.at[ri[s4].at[b]], rw[s2].at[b],
                                  sc[s2]).wait()
            pltpu.async_copy(rw[s2].at[b], m_sm.at[ci[s4].at[b]], sd[s2],
                             add=True)

    def drain(s4, s2):
        for b in range(GM):
            pltpu.make_async_copy(rw[s2].at[b], m_sm.at[ci[s4].at[b]],
                                  sd[s2]).wait()

    def turn(g, s4, do_drain=True, stage_ahead=True):
        if do_drain:
            drain((s4 + 2) % 4, g_s2(g))
        if stage_ahead:
            stage(g + 2, (s4 + 2) % 4)
        wait_stage(g, s4)
        fire(s4, g_s2(g))

    def g_s2(g):
        return g % 2 if isinstance(g, int) else 0

    stage(0, 0)
    stage(1, 1)
    turn(0, 0, do_drain=False)
    turn(1, 1, do_drain=False)
    turn(2, 2)
    turn(3, 3)

    def quad(q, _):
        g = q * 4
        for j in range(4):
            if True:
                gj = g + j
                drain((j + 2) % 4, j % 2)
                stage(gj + 2, (j + 2) % 4)
                wait_stage(gj, j % 4)
                fire(j % 4, j % 2)
        return 0

    lax.fori_loop(1, (NGM - 4) // 4, quad, 0)
    for j in range(4):
        g = NGM - 4 + j
        drain((j + 2) % 4, j % 2)
        if g + 2 < NGM:
            stage(g + 2, (j + 2) % 4)
        wait_stage(g, j % 4)
        fire(j % 4, j % 2)
    drain(2, 0)
    drain(3, 1)

    c0 = tb + NGM * GM
    pltpu.sync_copy(row2d_hbm.at[pl.ds(c0, 1), :], ri[0].at[pl.ds(0, 1), :])
    pltpu.sync_copy(col2d_hbm.at[pl.ds(c0, 1), :], ci[0].at[pl.ds(0, 1), :])
    pltpu.async_copy(embs_hbm.at[ri[0].at[0]], rw[0].at[0], sc[0]).wait()
    pltpu.sync_copy(rw[0].at[0], m_sm.at[ci[0].at[0]], add=True)

    @pl.when(wid < TAIL)
    def _tail():
        c1 = NW * FULL + wid
        pltpu.sync_copy(row2d_hbm.at[pl.ds(c1, 1), :], ri[1].at[pl.ds(0, 1), :])
        pltpu.sync_copy(col2d_hbm.at[pl.ds(c1, 1), :], ci[1].at[pl.ds(0, 1), :])
        pltpu.async_copy(embs_hbm.at[ri[1].at[0]], rw[1].at[0], sc[1]).wait()
        pltpu.sync_copy(rw[1].at[0], m_sm.at[ci[1].at[0]], add=True)

    plsc.subcore_barrier()
    pltpu.sync_copy(m_sm.at[pl.ds(sid * SEG, SEG)],
                    m_out.at[cid, pl.ds(sid * SEG, SEG)])


@functools.partial(
    pl.kernel,
    out_type=jax.ShapeDtypeStruct((N, D), jnp.float32),
    mesh=plsc.VectorSubcoreMesh(**_MESH),
    compiler_params=pltpu.CompilerParams(use_tc_tiling_on_sc=False),
    scratch_types=[
        pltpu.VMEM((RPT, D), jnp.float32),
        pltpu.VMEM((RPT, D), jnp.float32),
        pltpu.VMEM((RPT,), jnp.float32),
    ],
)
def _combine_sc(m_hbm, dis_hbm, out_hbm, m0v, m1v, disv):
    wid = _wid()
    base = wid * RPT
    pltpu.sync_copy(dis_hbm.at[pl.ds(base, RPT)], disv)

    def comb_rows(nrows):
        pltpu.sync_copy(m_hbm.at[0, pl.ds(base, nrows)],
                        m0v.at[pl.ds(0, nrows)])
        pltpu.sync_copy(m_hbm.at[1, pl.ds(base, nrows)],
                        m1v.at[pl.ds(0, nrows)])

        def rbody(i, _):
            dv = disv[pl.ds(i * L, L)]
            for j in range(L):
                r = i * L + j
                dspl = jnp.broadcast_to(dv[j], (L,))
                m0v[r, :] = dspl * (m0v[r, :] + m1v[r, :])
            return 0

        lax.fori_loop(0, nrows // L, rbody, 0)
        pltpu.sync_copy(m0v.at[pl.ds(0, nrows)],
                        out_hbm.at[pl.ds(base, nrows)])

    @pl.when(wid < NW - 1)
    def _full():
        comb_rows(RPT)

    @pl.when(wid == NW - 1)
    def _last():
        comb_rows(LASTV)


def kernel(edge_index, embedding):
    ei3 = edge_index.reshape(2, NCHUNK, CH)
    row2d = ei3[0]
    col2d = ei3[1]
    hist = _hist_kernel(row2d)
    dis, embs = _scale_sc(hist, embedding)
    m = _msg_kernel(row2d, col2d, embs)
    return _combine_sc(m, dis)


# speedup vs baseline: 1.0270x; 1.0270x over previous
import functools

import jax
import jax.numpy as jnp
from jax import lax
from jax.experimental import pallas as pl
from jax.experimental.pallas import tpu as pltpu
from jax.experimental.pallas import tpu_sc as plsc

N = 100000
E = 3200000
D = 16
NC = 2
NS = 16
L = 16
NW = NC * NS
NPAD = 102400
SEG = NPAD // NS
CH = 128
NCHUNK = E // CH
FULL = NCHUNK // NW
TAIL = NCHUNK - FULL * NW
G = 11
NG = FULL // G
NB = 4
GM = 5
NGM = (FULL - 1) // GM
RB = 1024
RB2 = 1000

_MESH = dict(core_axis_name="c", subcore_axis_name="s")


def _wid():
    return lax.axis_index("s") * NC + lax.axis_index("c")


@functools.partial(
    pl.kernel,
    out_type=jax.ShapeDtypeStruct((NC, NPAD), jnp.float32),
    mesh=plsc.VectorSubcoreMesh(**_MESH),
    compiler_params=pltpu.CompilerParams(use_tc_tiling_on_sc=False),
    scratch_types=(
        [pltpu.VMEM_SHARED((NPAD,), jnp.float32),
         pltpu.VMEM((SEG,), jnp.float32),
         pltpu.VMEM((CH,), jnp.float32)]
        + [pltpu.VMEM((G, CH), jnp.int32) for _ in range(NB)]
        + [pltpu.SemaphoreType.DMA for _ in range(2 * NB)]
    ),
)
def _hist_kernel(ei_hbm, hist_out, hist_sm, zbuf, ones, *rest):
    ri = rest[0:NB]
    si = rest[NB:2 * NB]
    sd = rest[2 * NB:3 * NB]
    cid = lax.axis_index("c")
    sid = lax.axis_index("s")
    wid = _wid()
    tb = wid * FULL

    zv = jnp.zeros((L,), jnp.float32)

    def zfill(i, _):
        zbuf[pl.ds(i * L, L)] = zv
        return 0

    lax.fori_loop(0, SEG // L, zfill, 0)

    ov = jnp.ones((L,), jnp.float32)

    def ofill(i, _):
        ones[pl.ds(i * L, L)] = ov
        return 0

    lax.fori_loop(0, CH // L, ofill, 0)

    pltpu.sync_copy(zbuf, hist_sm.at[pl.ds(sid * SEG, SEG)])
    plsc.subcore_barrier()

    def stage(g, s):
        pltpu.async_copy(ei_hbm.at[0, pl.ds(tb + g * G, G), :], ri[s], si[s])

    def wait_stage(g, s):
        pltpu.make_async_copy(
            ei_hbm.at[0, pl.ds(tb + g * G, G), :], ri[s], si[s]).wait()

    def fire(g, s):
        for b in range(G):
            pltpu.async_copy(ones, hist_sm.at[ri[s].at[b]], sd[s], add=True)

    def drain(s):
        for b in range(G):
            pltpu.make_async_copy(ones, hist_sm.at[ri[s].at[b]], sd[s]).wait()

    def turn(g, s, do_drain=True, stage_ahead=True):
        if do_drain:
            drain((s + 2) % NB)
        if stage_ahead:
            stage(g + 2, (s + 2) % NB)
        wait_stage(g, s)
        fire(g, s)

    stage(0, 0)
    stage(1, 1)
    turn(0, 0, do_drain=False)
    turn(1, 1, do_drain=False)
    turn(2, 2)
    turn(3, 3)

    def quad(q, _):
        g = q * NB
        for j in range(NB):
            turn(g + j, j)
        return 0

    lax.fori_loop(1, (NG - 3) // NB, quad, 0)
    turn(NG - 3, (NG - 3) % NB)
    turn(NG - 2, (NG - 2) % NB, stage_ahead=False)
    turn(NG - 1, (NG - 1) % NB, stage_ahead=False)
    drain((NG - 2) % NB)
    drain((NG - 1) % NB)

    @pl.when(wid < TAIL)
    def _tail():
        c0 = NW * FULL + wid
        pltpu.sync_copy(ei_hbm.at[0, pl.ds(c0, 1), :], ri[0].at[pl.ds(0, 1), :])
        pltpu.sync_copy(ones, hist_sm.at[ri[0].at[0]], add=True)

    plsc.subcore_barrier()
    pltpu.sync_copy(hist_sm.at[pl.ds(sid * SEG, SEG)],
                    hist_out.at[cid, pl.ds(sid * SEG, SEG)])


def _rsqrt(x):
    i = lax.bitcast_convert_type(x, jnp.int32)
    i = 0x5F3759DF - lax.shift_right_logical(i, 1)
    y = lax.bitcast_convert_type(i, jnp.float32)
    for _ in range(3):
        y = y * (1.5 - 0.5 * x * y * y)
    return y


RPT = NPAD // NW
LASTV = N - (NW - 1) * RPT


@functools.partial(
    pl.kernel,
    out_type=(jax.ShapeDtypeStruct((NPAD,), jnp.float32),
              jax.ShapeDtypeStruct((N, D), jnp.float32)),
    mesh=plsc.VectorSubcoreMesh(**_MESH),
    compiler_params=pltpu.CompilerParams(use_tc_tiling_on_sc=False),
    scratch_types=[
        pltpu.VMEM((RPT,), jnp.float32),
        pltpu.VMEM((RPT,), jnp.float32),
        pltpu.VMEM((RPT,), jnp.float32),
        pltpu.VMEM((RPT, D), jnp.float32),
    ],
)
def _scale_sc(hist_hbm, emb_hbm, dis_out, embs_out, h0, h1, disv, embv):
    wid = _wid()
    base = wid * RPT
    pltpu.sync_copy(hist_hbm.at[0, pl.ds(base, RPT)], h0)
    pltpu.sync_copy(hist_hbm.at[1, pl.ds(base, RPT)], h1)

    def dbody(i, _):
        deg = h0[pl.ds(i * L, L)] + h1[pl.ds(i * L, L)] + 1.0
        disv[pl.ds(i * L, L)] = _rsqrt(deg)
        return 0

    lax.fori_loop(0, RPT // L, dbody, 0)
    pltpu.sync_copy(disv, dis_out.at[pl.ds(base, RPT)])

    def scale_rows(nrows):
        pltpu.sync_copy(emb_hbm.at[pl.ds(base, nrows)],
                        embv.at[pl.ds(0, nrows)])

        def rbody(i, _):
            dv = disv[pl.ds(i * L, L)]
            for j in range(L):
                r = i * L + j
                dspl = jnp.broadcast_to(dv[j], (L,))
                embv[r, :] = embv[r, :] * dspl
            return 0

        lax.fori_loop(0, nrows // L, rbody, 0)
        pltpu.sync_copy(embv.at[pl.ds(0, nrows)],
                        embs_out.at[pl.ds(base, nrows)])

    @pl.when(wid < NW - 1)
    def _full():
        scale_rows(RPT)

    @pl.when(wid == NW - 1)
    def _last():
        scale_rows(LASTV)


@functools.partial(
    pl.kernel,
    out_type=jax.ShapeDtypeStruct((NC, NPAD, D), jnp.float32),
    mesh=plsc.VectorSubcoreMesh(**_MESH),
    compiler_params=pltpu.CompilerParams(use_tc_tiling_on_sc=False),
    scratch_types=(
        [pltpu.VMEM_SHARED((NPAD, D), jnp.float32)]
        + [pltpu.VMEM((GM, CH), jnp.int32) for _ in range(8)]
        + [pltpu.VMEM((GM, CH, D), jnp.float32) for _ in range(2)]
        + [pltpu.SemaphoreType.DMA for _ in range(8)]
    ),
)
def _msg_kernel(ei_hbm, embs_hbm, m_out, m_sm, *rest):
    ri = rest[0:4]
    ci = rest[4:8]
    rw = rest[8:10]
    si = rest[10:14]
    sc = rest[14:16]
    sd = rest[16:18]
    cid = lax.axis_index("c")
    sid = lax.axis_index("s")
    wid = _wid()
    tb = wid * FULL

    @pl.when(cid == 0)
    def _init0():
        nv = N // NS
        pltpu.sync_copy(embs_hbm.at[pl.ds(sid * nv, nv)],
                        m_sm.at[pl.ds(sid * nv, nv)])

    @pl.when(cid == 1)
    def _init1():
        zv = jnp.zeros((L,), jnp.float32)
        for b in range(GM):
            def zfill(r, _, _b=b):
                rw[0][_b, r, :] = zv
                return 0
            lax.fori_loop(0, CH, zfill, 0)
        def zcopy(j, _):
            pltpu.async_copy(rw[0].at[0],
                             m_sm.at[pl.ds(sid * SEG + j * CH, CH)], sd[0])
            return 0

        lax.fori_loop(0, SEG // CH, zcopy, 0)

        def zwait(j, _):
            pltpu.make_async_copy(
                rw[0].at[0], m_sm.at[pl.ds(sid * SEG + j * CH, CH)],
                sd[0]).wait()
            return 0

        lax.fori_loop(0, SEG // CH, zwait, 0)

    plsc.subcore_barrier()

    def stage(g, s4):
        c0 = tb + g * GM
        pltpu.async_copy(ei_hbm.at[0, pl.ds(c0, GM), :], ri[s4], si[s4])
        pltpu.async_copy(ei_hbm.at[1, pl.ds(c0, GM), :], ci[s4], si[s4])

    def wait_stage(g, s4):
        c0 = tb + g * GM
        pltpu.make_async_copy(ei_hbm.at[0, pl.ds(c0, GM), :], ri[s4],
                              si[s4]).wait()
        pltpu.make_async_copy(ei_hbm.at[1, pl.ds(c0, GM), :], ci[s4],
                              si[s4]).wait()

    def fire(s4, s2):
        for b in range(GM):
            pltpu.async_copy(embs_hbm.at[ri[s4].at[b]], rw[s2].at[b], sc[s2])
        for b in range(GM):
            pltpu.make_async_copy(embs_hbm.at[ri[s4].at[b]], rw[s2].at[b],
                                  sc[s2]).wait()
            pltpu.async_copy(rw[s2].at[b], m_sm.at[ci[s4].at[b]], sd[s2],
                             add=True)

    def drain(s4, s2):
        for b in range(GM):
            pltpu.make_async_copy(rw[s2].at[b], m_sm.at[ci[s4].at[b]],
                                  sd[s2]).wait()

    def turn(g, s4, do_drain=True, stage_ahead=True):
        if do_drain:
            drain((s4 + 2) % 4, g_s2(g))
        if stage_ahead:
            stage(g + 2, (s4 + 2) % 4)
        wait_stage(g, s4)
        fire(s4, g_s2(g))

    def g_s2(g):
        return g % 2 if isinstance(g, int) else 0

    stage(0, 0)
    stage(1, 1)
    turn(0, 0, do_drain=False)
    turn(1, 1, do_drain=False)
    turn(2, 2)
    turn(3, 3)

    def quad(q, _):
        g = q * 4
        for j in range(4):
            if True:
                gj = g + j
                drain((j + 2) % 4, j % 2)
                stage(gj + 2, (j + 2) % 4)
                wait_stage(gj, j % 4)
                fire(j % 4, j % 2)
        return 0

    lax.fori_loop(1, (NGM - 4) // 4, quad, 0)
    for j in range(4):
        g = NGM - 4 + j
        drain((j + 2) % 4, j % 2)
        if g + 2 < NGM:
            stage(g + 2, (j + 2) % 4)
        wait_stage(g, j % 4)
        fire(j % 4, j % 2)
    drain(2, 0)
    drain(3, 1)

    c0 = tb + NGM * GM
    pltpu.sync_copy(ei_hbm.at[0, pl.ds(c0, 1), :], ri[0].at[pl.ds(0, 1), :])
    pltpu.sync_copy(ei_hbm.at[1, pl.ds(c0, 1), :], ci[0].at[pl.ds(0, 1), :])
    pltpu.async_copy(embs_hbm.at[ri[0].at[0]], rw[0].at[0], sc[0]).wait()
    pltpu.sync_copy(rw[0].at[0], m_sm.at[ci[0].at[0]], add=True)

    @pl.when(wid < TAIL)
    def _tail():
        c1 = NW * FULL + wid
        pltpu.sync_copy(ei_hbm.at[0, pl.ds(c1, 1), :], ri[1].at[pl.ds(0, 1), :])
        pltpu.sync_copy(ei_hbm.at[1, pl.ds(c1, 1), :], ci[1].at[pl.ds(0, 1), :])
        pltpu.async_copy(embs_hbm.at[ri[1].at[0]], rw[1].at[0], sc[1]).wait()
        pltpu.sync_copy(rw[1].at[0], m_sm.at[ci[1].at[0]], add=True)

    plsc.subcore_barrier()
    pltpu.sync_copy(m_sm.at[pl.ds(sid * SEG, SEG)],
                    m_out.at[cid, pl.ds(sid * SEG, SEG)])


@functools.partial(
    pl.kernel,
    out_type=jax.ShapeDtypeStruct((N, D), jnp.float32),
    mesh=plsc.VectorSubcoreMesh(**_MESH),
    compiler_params=pltpu.CompilerParams(use_tc_tiling_on_sc=False),
    scratch_types=[
        pltpu.VMEM((RPT, D), jnp.float32),
        pltpu.VMEM((RPT, D), jnp.float32),
        pltpu.VMEM((RPT,), jnp.float32),
    ],
)
def _combine_sc(m_hbm, dis_hbm, out_hbm, m0v, m1v, disv):
    wid = _wid()
    base = wid * RPT
    pltpu.sync_copy(dis_hbm.at[pl.ds(base, RPT)], disv)

    def comb_rows(nrows):
        pltpu.sync_copy(m_hbm.at[0, pl.ds(base, nrows)],
                        m0v.at[pl.ds(0, nrows)])
        pltpu.sync_copy(m_hbm.at[1, pl.ds(base, nrows)],
                        m1v.at[pl.ds(0, nrows)])

        def rbody(i, _):
            dv = disv[pl.ds(i * L, L)]
            for j in range(L):
                r = i * L + j
                dspl = jnp.broadcast_to(dv[j], (L,))
                m0v[r, :] = dspl * (m0v[r, :] + m1v[r, :])
            return 0

        lax.fori_loop(0, nrows // L, rbody, 0)
        pltpu.sync_copy(m0v.at[pl.ds(0, nrows)],
                        out_hbm.at[pl.ds(base, nrows)])

    @pl.when(wid < NW - 1)
    def _full():
        comb_rows(RPT)

    @pl.when(wid == NW - 1)
    def _last():
        comb_rows(LASTV)


def kernel(edge_index, embedding):
    ei3 = edge_index.reshape(2, NCHUNK, CH)
    hist = _hist_kernel(ei3)
    dis, embs = _scale_sc(hist, embedding)
    m = _msg_kernel(ei3, embs)
    return _combine_sc(m, dis)
